# Initial kernel scaffold; baseline (speedup 1.0000x reference)
import jax
import jax.numpy as jnp
from jax.experimental import pallas as pl


def kernel(x, edge_index, batch, cls_embed, mean_embed, params):
    raise NotImplementedError("write your pallas kernel here")


# speedup vs baseline: 23.3467x; 23.3467x over previous
import functools

import jax
import jax.numpy as jnp
from jax import lax
from jax.experimental import pallas as pl
from jax.experimental.pallas import tpu as pltpu
from jax.experimental.pallas import tpu_sc as plsc

N_NODES = 10000
N_GRAPHS = 64
NP = 10240
EPW = 10240
K = 128
NCHUNK = EPW // K
EP = EPW * 32

_f32 = jnp.float32
_i32 = jnp.int32


def _make_edge_pass(F1, F2, has_mpnn):
    F = F1 + F2
    two = F2 > 0

    mesh = plsc.VectorSubcoreMesh(core_axis_name="c", subcore_axis_name="s",
                                  num_cores=2, num_subcores=16)
    out_type = [jax.ShapeDtypeStruct((2, NP, F), _f32),
                jax.ShapeDtypeStruct((2, NP), _f32)]
    if two:
        out_type.append(jax.ShapeDtypeStruct((2, NP), _f32))
    if has_mpnn:
        out_type.append(jax.ShapeDtypeStruct((2, NP), _f32))
        out_type.append(jax.ShapeDtypeStruct((2, NP, 64), _f32))

    scratch = [
        pltpu.VMEM((K,), _i32),
        pltpu.VMEM((K,), _i32),
        pltpu.VMEM((K, F), _f32),
        pltpu.VMEM((K,), _f32),
        pltpu.VMEM((K,), _f32),
        pltpu.SemaphoreType.DMA,
        pltpu.SemaphoreType.DMA,
        pltpu.SemaphoreType.DMA,
        pltpu.VMEM_SHARED((NP, F), _f32),
        pltpu.VMEM_SHARED((NP,), _f32),
    ]
    if two:
        scratch += [
            pltpu.VMEM((K,), _f32),
            pltpu.VMEM((K,), _f32),
            pltpu.SemaphoreType.DMA,
            pltpu.SemaphoreType.DMA,
            pltpu.VMEM_SHARED((NP,), _f32),
        ]
    if has_mpnn:
        scratch += [
            pltpu.VMEM((K, 64), _f32),
            pltpu.VMEM((K,), _f32),
            pltpu.SemaphoreType.DMA,
            pltpu.VMEM_SHARED((NP,), _f32),
            pltpu.VMEM_SHARED((NP, 64), _f32),
        ]

    @functools.partial(
        pl.kernel, out_type=tuple(out_type), mesh=mesh,
        scratch_types=tuple(scratch),
        compiler_params=pltpu.CompilerParams(use_tc_tiling_on_sc=False))
    def run(*refs):
        it = iter(refs)
        src_r = next(it); dst_r = next(it); tab_r = next(it)
        ss1_r = next(it); sd1_r = next(it)
        if two:
            ss2_r = next(it); sd2_r = next(it)
        if has_mpnn:
            xm_r = next(it)
        oacc_r = next(it); ow1_r = next(it)
        if two:
            ow2_r = next(it)
        if has_mpnn:
            odeg_r = next(it); om_r = next(it)
        sidx = next(it); didx = next(it); rowb = next(it)
        w1b = next(it); d1b = next(it)
        sem_row = next(it); sem_s1 = next(it); sem_d1 = next(it)
        acc = next(it); accw1 = next(it)
        if two:
            w2b = next(it); d2b = next(it)
            sem_s2 = next(it); sem_d2 = next(it)
            accw2 = next(it)
        if has_mpnn:
            mrow = next(it); oneb = next(it); sem_m = next(it)
            accd = next(it); accm = next(it)

        c = lax.axis_index("c")
        s = lax.axis_index("s")
        wrk = c * 16 + s
        zero16 = jnp.zeros((16,), _f32)

        for i in range(K):
            for b in range(F // 16):
                rowb[i, pl.ds(b * 16, 16)] = zero16
            if has_mpnn:
                for b in range(4):
                    mrow[i, pl.ds(b * 16, 16)] = zero16
        for g in range(K // 16):
            w1b[pl.ds(g * 16, 16)] = zero16
            if has_mpnn:
                oneb[pl.ds(g * 16, 16)] = jnp.full((16,), 1.0, _f32)
        r0 = s * (NP // 16)
        nslab = NP // 16 // K
        for i in range(nslab):
            rr = r0 + i * K
            pltpu.sync_copy(rowb, acc.at[pl.ds(rr, K)])
            pltpu.sync_copy(w1b, accw1.at[pl.ds(rr, K)])
            if two:
                pltpu.sync_copy(w1b, accw2.at[pl.ds(rr, K)])
            if has_mpnn:
                pltpu.sync_copy(mrow, accm.at[pl.ds(rr, K)])
                pltpu.sync_copy(w1b, accd.at[pl.ds(rr, K)])
        plsc.subcore_barrier()

        ebase = wrk * EPW

        def chunk(i, carry):
            off = ebase + i * K
            pltpu.sync_copy(src_r.at[pl.ds(off, K)], sidx)
            pltpu.sync_copy(dst_r.at[pl.ds(off, K)], didx)
            cr = pltpu.async_copy(tab_r.at[sidx], rowb, sem_row)
            c1 = pltpu.async_copy(ss1_r.at[sidx], w1b, sem_s1)
            cd1 = pltpu.async_copy(sd1_r.at[didx], d1b, sem_d1)
            if two:
                c2 = pltpu.async_copy(ss2_r.at[sidx], w2b, sem_s2)
                cd2 = pltpu.async_copy(sd2_r.at[didx], d2b, sem_d2)
            if has_mpnn:
                cm = pltpu.async_copy(xm_r.at[sidx], mrow, sem_m)
            c1.wait(); cd1.wait()
            if two:
                c2.wait(); cd2.wait()
            cr.wait()

            def group(g, carry2):
                gb = g * 16
                z1 = w1b[pl.ds(gb, 16)] + d1b[pl.ds(gb, 16)]
                w1 = jnp.exp(jnp.where(z1 > 0, z1, z1 * 0.2))
                w1b[pl.ds(gb, 16)] = w1
                if two:
                    z2 = w2b[pl.ds(gb, 16)] + d2b[pl.ds(gb, 16)]
                    w2 = jnp.exp(jnp.where(z2 > 0, z2, z2 * 0.2))
                    w2b[pl.ds(gb, 16)] = w2
                for j in range(16):
                    e = gb + j
                    w1s = w1[j]
                    for b in range(F1 // 16):
                        rowb[e, pl.ds(b * 16, 16)] = rowb[e, pl.ds(b * 16, 16)] * w1s
                    if two:
                        w2s = w2[j]
                        for b in range(F2 // 16):
                            col = F1 + b * 16
                            rowb[e, pl.ds(col, 16)] = rowb[e, pl.ds(col, 16)] * w2s
                return carry2

            lax.fori_loop(0, K // 16, group, 0)

            pltpu.sync_copy(rowb, acc.at[didx], add=True)
            pltpu.sync_copy(w1b, accw1.at[didx], add=True)
            if two:
                pltpu.sync_copy(w2b, accw2.at[didx], add=True)
            if has_mpnn:
                cm.wait()
                pltpu.sync_copy(mrow, accm.at[didx], add=True)
                pltpu.sync_copy(oneb, accd.at[didx], add=True)
            return carry

        lax.fori_loop(0, NCHUNK, chunk, 0)
        plsc.subcore_barrier()

        for i in range(nslab):
            rr = r0 + i * K
            pltpu.sync_copy(acc.at[pl.ds(rr, K)], rowb)
            pltpu.sync_copy(rowb, oacc_r.at[c, pl.ds(rr, K)])
            pltpu.sync_copy(accw1.at[pl.ds(rr, K)], w1b)
            pltpu.sync_copy(w1b, ow1_r.at[c, pl.ds(rr, K)])
            if two:
                pltpu.sync_copy(accw2.at[pl.ds(rr, K)], w1b)
                pltpu.sync_copy(w1b, ow2_r.at[c, pl.ds(rr, K)])
            if has_mpnn:
                pltpu.sync_copy(accd.at[pl.ds(rr, K)], w1b)
                pltpu.sync_copy(w1b, odeg_r.at[c, pl.ds(rr, K)])
                pltpu.sync_copy(accm.at[pl.ds(rr, K)], mrow)
                pltpu.sync_copy(mrow, om_r.at[c, pl.ds(rr, K)])

    return run


_make_edge_pass_cached = functools.cache(_make_edge_pass)


def _edge_pass1(*args):
    return _make_edge_pass_cached(F1=16, F2=16, has_mpnn=True)(*args)


def _edge_pass2(*args):
    return _make_edge_pass_cached(F1=64, F2=16, has_mpnn=False)(*args)


def _edge_pass3(*args):
    return _make_edge_pass_cached(F1=64, F2=0, has_mpnn=False)(*args)


def _leaky(z):
    return jnp.where(z > 0, z, z * 0.2)


def _tc_prep(x_r, w96_r, ag_r,
             tab_r, ssg_r, ssd_r, sdg_r, sdd_r, xm_r):
    h = jnp.dot(x_r[...], w96_r[...], preferred_element_type=_f32)
    hg = h[:, 0:16]
    hd = h[:, 16:32]
    ag = ag_r[...]
    tab_r[...] = h[:, 0:32]
    xm_r[...] = h[:, 32:96]
    ssg_r[...] = jnp.sum(hg * ag[0:1, :], axis=1)
    sdg_r[...] = jnp.sum(hg * ag[1:2, :], axis=1)
    ssd_r[...] = jnp.sum(hd * ag[2:3, :], axis=1)
    sdd_r[...] = jnp.sum(hd * ag[3:4, :], axis=1)


def _tc_mid_a(p1a_r, p1b_r, w1a_r, w1b_r, w2a_r, w2b_r, dega_r, degb_r,
              pma_r, pmb_r, tab1_r, ssg_r, sdg_r, ssd_r, sdd_r,
              bg1_r, bd1_r, wm2_r, bm_r,
              g_r, h0_r, mp_r):
    acc = p1a_r[...] + p1b_r[...]
    sw1 = w1a_r[...] + w1b_r[...]
    sw2 = w2a_r[...] + w2b_r[...]
    t1 = tab1_r[...]
    hg1 = t1[:, 0:16]
    hd1 = t1[:, 16:32]
    wg = jnp.exp(_leaky(ssg_r[...] + sdg_r[...]))
    wd = jnp.exp(_leaky(ssd_r[...] + sdd_r[...]))
    g1 = (acc[:, 0:16] + wg * hg1) / (sw1 + wg) + bg1_r[...]
    d1 = (acc[:, 16:32] + wd * hd1) / (sw2 + wd) + bd1_r[...]
    g_r[...] = jnp.maximum(g1, 0.0)
    h0_r[...] = d1
    deg = dega_r[...] + degb_r[...]

    accm = pma_r[...] + pmb_r[...]
    bm = bm_r[...]
    agg = accm + deg * bm[0:1, :]
    mp_r[...] = jnp.dot(agg, wm2_r[...], preferred_element_type=_f32) + bm[1:2, :]


def _tc_mid_b(g_r, h0f_r, h0_r, bn_r, wg2_r, a2_r, wd2_r,
              tab2_r, ssg2_r, ssd2_r, sdg2_r, sdd2_r):
    d1f = h0f_r[...]
    d1 = h0_r[...]
    rows = lax.broadcasted_iota(_i32, (NP, 1), 0)
    mask = (rows < N_NODES).astype(_f32)
    dm = d1f * mask
    inv_n = 1.0 / N_NODES
    mu = jnp.sum(dm, axis=0, keepdims=True) * inv_n
    ex2 = jnp.sum(dm * d1f, axis=0, keepdims=True) * inv_n
    var = ex2 - mu * mu
    bn = bn_r[...]
    dn = (d1 - mu) * jax.lax.rsqrt(var + 1e-5) * bn[0:1, :] + bn[1:2, :]
    dn = jnp.maximum(dn, 0.0)

    hg2 = jnp.dot(g_r[...], wg2_r[...], preferred_element_type=_f32)
    a2 = a2_r[...]
    hd2 = jnp.dot(dn, wd2_r[...], preferred_element_type=_f32)
    tab2_r[...] = jnp.concatenate([hg2, hd2], axis=1)
    ssg2_r[...] = jnp.sum(hg2 * a2[0:1, :], axis=1)
    sdg2_r[...] = jnp.sum(hg2 * a2[1:2, :], axis=1)
    ssd2_r[...] = jnp.sum(hd2 * a2[2:3, 0:16], axis=1)
    sdd2_r[...] = jnp.sum(hd2 * a2[3:4, 0:16], axis=1)


def _tc_mid2(p2a_r, p2b_r, w1a_r, w1b_r, w2a_r, w2b_r, tab2_r,
             ssg_r, sdg_r, ssd_r, sdd_r,
             bg2_r, h0_r, bd2_r, wd3_r, a3_r,
             g_r, tab3_r, ss3_r, sd3_r):
    acc = p2a_r[...] + p2b_r[...]
    sw1 = w1a_r[...] + w1b_r[...]
    sw2 = w2a_r[...] + w2b_r[...]
    t2 = tab2_r[...]
    hg2 = t2[:, 0:64]
    hd2 = t2[:, 64:80]
    wg = jnp.exp(_leaky(ssg_r[...] + sdg_r[...]))
    wd = jnp.exp(_leaky(ssd_r[...] + sdd_r[...]))
    g_r[...] = (acc[:, 0:64] + wg * hg2) / (sw1 + wg) + bg2_r[...]
    d2 = (acc[:, 64:80] + wd * hd2) / (sw2 + wd) + bd2_r[...] + h0_r[...]

    hd3 = jnp.dot(d2, wd3_r[...], preferred_element_type=_f32)
    a3 = a3_r[...]
    tab3_r[...] = hd3
    ss3_r[...] = jnp.sum(hd3 * a3[0:1, :], axis=1)
    sd3_r[...] = jnp.sum(hd3 * a3[1:2, :], axis=1)


def _tc_fin(p3a_r, p3b_r, w1a_r, w1b_r, tab3_r, ss3_r, sd3_r, bd3_r, d_r):
    acc = p3a_r[...] + p3b_r[...]
    sw = w1a_r[...] + w1b_r[...]
    h3 = tab3_r[...]
    w = jnp.exp(_leaky(ss3_r[...] + sd3_r[...]))
    d_r[...] = (acc + w * h3) / (sw + w) + bd3_r[...]


_BLK = 2048
_NB = NP // _BLK


def _tc_pool(g_r, d_r, mp_r, batch_r, out_r):
    i = pl.program_id(0)
    gids = lax.broadcasted_iota(_i32, (N_GRAPHS, _BLK), 0)
    P = (gids == batch_r[...]).astype(_f32)
    part = jnp.concatenate([
        jnp.dot(P, g_r[...], preferred_element_type=_f32),
        jnp.dot(P, d_r[...], preferred_element_type=_f32),
        jnp.dot(P, mp_r[...], preferred_element_type=_f32),
        jnp.sum(P, axis=1, keepdims=True),
        jnp.zeros((N_GRAPHS, 63), _f32)], axis=1)

    @pl.when(i == 0)
    def _():
        out_r[...] = jnp.zeros((N_GRAPHS, 256), _f32)

    out_r[...] += part


def _tc_head(pool_r, cls_r, mean_r, wc1_r, wc2_r, wm1_r, wm2_r, wf1_r,
             wf2_r, wa1_r, wa2_r, bvec_r, out_r):
    pool = pool_r[...]
    cnt = jnp.maximum(pool[:, 192:193], 1.0)
    gp = pool[:, 0:64] / cnt
    dp = pool[:, 64:128] / cnt
    mpp = pool[:, 128:192] / cnt

    bv = bvec_r[...]
    c = jnp.maximum(jnp.dot(cls_r[...], wc1_r[...], preferred_element_type=_f32)
                    + bv[0:1, 0:128], 0.0)
    c = jnp.dot(c, wc2_r[...], preferred_element_type=_f32) + bv[1:2, 0:64]
    me = jnp.maximum(jnp.dot(mean_r[...], wm1_r[...], preferred_element_type=_f32)
                     + bv[2:3, 0:128], 0.0)
    me = jnp.dot(me, wm2_r[...], preferred_element_type=_f32) + bv[3:4, 0:64]
    emb = jnp.maximum(jnp.dot(jnp.concatenate([c, me], axis=1), wf1_r[...],
                              preferred_element_type=_f32) + bv[4:5, 0:32], 0.0)
    emb = jnp.dot(emb, wf2_r[...], preferred_element_type=_f32) + bv[5:6, 0:64]

    feat = jnp.concatenate([gp, dp, mpp, emb], axis=1)
    o = jnp.maximum(jnp.dot(feat, wa1_r[...], preferred_element_type=_f32)
                    + bv[6:7, 0:64], 0.0)
    out_r[...] = jnp.dot(o, wa2_r[...], preferred_element_type=_f32) + bv[7:8, 0:1]


_PALLAS_CALL = pl.pallas_call


def _spec(kind, shape):
    nd = len(shape)
    if kind == 'R':
        return pl.BlockSpec((_BLK,) + tuple(shape[1:]),
                            lambda i: (i,) + (0,) * (nd - 1))
    if kind == 'r':
        return pl.BlockSpec((_BLK,), lambda i: (i,))
    if kind == 'b':
        return pl.BlockSpec((1, _BLK), lambda i: (0, i))
    return pl.BlockSpec(tuple(shape), lambda i: (0,) * nd)


def _tc_call(body, outs, args, grid=None):
    if grid is None:
        return _PALLAS_CALL(
            body,
            out_shape=tuple(jax.ShapeDtypeStruct(sh, _f32) for sh, _ in outs),
        )(*[v for v, _ in args])
    return _PALLAS_CALL(
        body,
        grid=(grid,),
        in_specs=[_spec(k, v.shape) for v, k in args],
        out_specs=tuple(_spec(k, sh) for sh, k in outs),
        out_shape=tuple(jax.ShapeDtypeStruct(sh, _f32) for sh, _ in outs),
    )(*[v for v, _ in args])


def kernel(x, edge_index, batch, cls_embed, mean_embed, params):
    p = params
    xp = jnp.pad(x.astype(_f32), ((0, NP - N_NODES), (0, 0)))
    pad_e = EP - edge_index.shape[1]
    src = jnp.concatenate([edge_index[0].astype(_i32),
                           jnp.full((pad_e,), N_NODES, _i32)])
    dst = jnp.concatenate([edge_index[1].astype(_i32),
                           jnp.full((pad_e,), N_NODES, _i32)])
    batch_p = jnp.pad(batch.astype(_i32), (0, NP - N_NODES),
                      constant_values=N_GRAPHS).reshape(1, NP)

    w96 = jnp.concatenate([p['W_g1'], p['W_d1'], p['W_m1']], axis=1)
    ag = jnp.stack([p['as_g1'], p['ad_g1'], p['as_d1'], p['ad_d1']])
    pad16 = jnp.zeros((48,), _f32)
    a2 = jnp.stack([p['as_g2'], p['ad_g2'],
                    jnp.concatenate([p['as_d2'], pad16]),
                    jnp.concatenate([p['ad_d2'], pad16])])
    a3 = jnp.stack([p['as_d3'], p['ad_d3']])
    bm = jnp.stack([p['b_m1'], p['b_m2']])
    bn = jnp.stack([p['bn_g'], p['bn_b']])

    def row128(v):
        return jnp.pad(v, (0, 128 - v.shape[0]))
    bvec = jnp.stack([row128(p['bc1']), row128(p['bc2']), row128(p['bm1']),
                      row128(p['bm2']), row128(p['bf1']), row128(p['bf2']),
                      row128(p['ba1']), row128(p['ba2'])])

    def col(v):
        return v.reshape(NP, 1)

    tab1, ssg1, ssd1, sdg1, sdd1, xm = _tc_call(
        _tc_prep,
        [((NP, 32), 'R'), ((NP,), 'r'), ((NP,), 'r'), ((NP,), 'r'),
         ((NP,), 'r'), ((NP, 64), 'R')],
        [(xp, 'R'), (w96, 'F'), (ag, 'F')], grid=_NB)

    acc1, ww1, ww2, wdeg, accm = _edge_pass1(
        src, dst, tab1, ssg1, sdg1, ssd1, sdd1, xm)

    g, h0, mp = _tc_call(
        _tc_mid_a, [((NP, 16), 'R'), ((NP, 16), 'R'), ((NP, 64), 'R')],
        [(acc1[0], 'R'), (acc1[1], 'R'),
         (col(ww1[0]), 'R'), (col(ww1[1]), 'R'),
         (col(ww2[0]), 'R'), (col(ww2[1]), 'R'),
         (col(wdeg[0]), 'R'), (col(wdeg[1]), 'R'),
         (accm[0], 'R'), (accm[1], 'R'), (tab1, 'R'),
         (col(ssg1), 'R'), (col(sdg1), 'R'), (col(ssd1), 'R'),
         (col(sdd1), 'R'),
         (p['b_g1'].reshape(1, 16), 'F'), (p['b_d1'].reshape(1, 16), 'F'),
         (p['W_m2'], 'F'), (bm, 'F')], grid=_NB)
    tab2, ssg2, ssd2, sdg2, sdd2 = _tc_call(
        _tc_mid_b,
        [((NP, 80), 'R'), ((NP,), 'r'), ((NP,), 'r'), ((NP,), 'r'),
         ((NP,), 'r')],
        [(g, 'R'), (h0, 'F'), (h0, 'R'), (bn, 'F'), (p['W_g2'], 'F'),
         (a2, 'F'), (p['W_d2'], 'F')], grid=_NB)

    acc2, w21, w22 = _edge_pass2(src, dst, tab2, ssg2, sdg2, ssd2, sdd2)

    gfin, tab3, ss3, sd3 = _tc_call(
        _tc_mid2,
        [((NP, 64), 'R'), ((NP, 64), 'R'), ((NP,), 'r'), ((NP,), 'r')],
        [(acc2[0], 'R'), (acc2[1], 'R'),
         (col(w21[0]), 'R'), (col(w21[1]), 'R'),
         (col(w22[0]), 'R'), (col(w22[1]), 'R'), (tab2, 'R'),
         (col(ssg2), 'R'), (col(sdg2), 'R'), (col(ssd2), 'R'),
         (col(sdd2), 'R'),
         (p['b_g2'].reshape(1, 64), 'F'), (h0, 'R'),
         (p['b_d2'].reshape(1, 16), 'F'), (p['W_d3'], 'F'),
         (a3, 'F')], grid=_NB)

    acc3, w31 = _edge_pass3(src, dst, tab3, ss3, sd3)

    dfin, = _tc_call(
        _tc_fin, [((NP, 64), 'R')],
        [(acc3[0], 'R'), (acc3[1], 'R'),
         (col(w31[0]), 'R'), (col(w31[1]), 'R'), (tab3, 'R'),
         (col(ss3), 'R'), (col(sd3), 'R'),
         (p['b_d3'].reshape(1, 64), 'F')], grid=_NB)
    pool, = _tc_call(
        _tc_pool, [((N_GRAPHS, 256), 'F')],
        [(gfin, 'R'), (dfin, 'R'), (mp, 'R'), (batch_p, 'b')], grid=_NB)
    out, = _tc_call(
        _tc_head, [((N_GRAPHS, 1), 'F')],
        [(pool, 'F'), (cls_embed, 'F'), (mean_embed, 'F'),
         (p['Wc1'], 'F'), (p['Wc2'], 'F'), (p['Wm1'], 'F'), (p['Wm2'], 'F'),
         (p['Wf1'], 'F'), (p['Wf2'], 'F'), (p['Wa1'], 'F'), (p['Wa2'], 'F'),
         (bvec, 'F')])
    return out


# speedup vs baseline: 38.1009x; 1.6320x over previous
import functools

import jax
import jax.numpy as jnp
from jax import lax
from jax.experimental import pallas as pl
from jax.experimental.pallas import tpu as pltpu
from jax.experimental.pallas import tpu_sc as plsc

N_NODES = 10000
N_GRAPHS = 64
NP = 10240
EPW = 10240
K = 128
NCHUNK = EPW // K
EP = EPW * 32

_f32 = jnp.float32
_i32 = jnp.int32


def _make_edge_pass(F1, F2, has_mpnn):
    F = F1 + F2
    two = F2 > 0

    mesh = plsc.VectorSubcoreMesh(core_axis_name="c", subcore_axis_name="s",
                                  num_cores=2, num_subcores=16)
    out_type = [jax.ShapeDtypeStruct((2, NP, F), _f32),
                jax.ShapeDtypeStruct((2, NP), _f32)]
    if two:
        out_type.append(jax.ShapeDtypeStruct((2, NP), _f32))
    if has_mpnn:
        out_type.append(jax.ShapeDtypeStruct((2, NP), _f32))
        out_type.append(jax.ShapeDtypeStruct((2, NP, 64), _f32))

    nbuf = 2
    scratch = []
    for _ in range(nbuf):
        scratch += [
            pltpu.VMEM((K,), _i32),
            pltpu.VMEM((K,), _i32),
            pltpu.VMEM((K, F), _f32),
            pltpu.VMEM((K,), _f32),
            pltpu.VMEM((K,), _f32),
        ]
        if two:
            scratch += [pltpu.VMEM((K,), _f32), pltpu.VMEM((K,), _f32)]
        if has_mpnn:
            scratch += [pltpu.VMEM((K, 64), _f32)]
        scratch += [pltpu.SemaphoreType.DMA, pltpu.SemaphoreType.DMA]
    scratch += [
        pltpu.VMEM_SHARED((NP, F), _f32),
        pltpu.VMEM_SHARED((NP,), _f32),
    ]
    if two:
        scratch += [pltpu.VMEM_SHARED((NP,), _f32)]
    if has_mpnn:
        scratch += [
            pltpu.VMEM((K,), _f32),
            pltpu.VMEM_SHARED((NP,), _f32),
            pltpu.VMEM_SHARED((NP, 64), _f32),
        ]

    @functools.partial(
        pl.kernel, out_type=tuple(out_type), mesh=mesh,
        scratch_types=tuple(scratch),
        compiler_params=pltpu.CompilerParams(use_tc_tiling_on_sc=False))
    def run(*refs):
        it = iter(refs)
        src_r = next(it); dst_r = next(it); tab_r = next(it)
        ss1_r = next(it); sd1_r = next(it)
        if two:
            ss2_r = next(it); sd2_r = next(it)
        if has_mpnn:
            xm_r = next(it)
        oacc_r = next(it); ow1_r = next(it)
        if two:
            ow2_r = next(it)
        if has_mpnn:
            odeg_r = next(it); om_r = next(it)
        sets = []
        for _ in range(nbuf):
            st = {}
            st['sidx'] = next(it); st['didx'] = next(it); st['rowb'] = next(it)
            st['w1b'] = next(it); st['d1b'] = next(it)
            if two:
                st['w2b'] = next(it); st['d2b'] = next(it)
            if has_mpnn:
                st['mrow'] = next(it)
            st['sem_g'] = next(it); st['sem_sc'] = next(it)
            sets.append(st)
        acc = next(it); accw1 = next(it)
        accw2 = next(it) if two else None
        if has_mpnn:
            oneb = next(it); accd = next(it); accm = next(it)

        c = lax.axis_index("c")
        s = lax.axis_index("s")
        wrk = c * 16 + s
        zero16 = jnp.zeros((16,), _f32)
        s0 = sets[0]

        rowb0 = s0['rowb']; w1b0 = s0['w1b']
        for i in range(K):
            for b in range(F // 16):
                rowb0[i, pl.ds(b * 16, 16)] = zero16
            if has_mpnn:
                for b in range(4):
                    s0['mrow'][i, pl.ds(b * 16, 16)] = zero16
        for g in range(K // 16):
            w1b0[pl.ds(g * 16, 16)] = zero16
            if has_mpnn:
                oneb[pl.ds(g * 16, 16)] = jnp.full((16,), 1.0, _f32)
        r0 = s * (NP // 16)
        nslab = NP // 16 // K
        for i in range(nslab):
            rr = r0 + i * K
            pltpu.sync_copy(rowb0, acc.at[pl.ds(rr, K)])
            pltpu.sync_copy(w1b0, accw1.at[pl.ds(rr, K)])
            if two:
                pltpu.sync_copy(w1b0, accw2.at[pl.ds(rr, K)])
            if has_mpnn:
                pltpu.sync_copy(s0['mrow'], accm.at[pl.ds(rr, K)])
                pltpu.sync_copy(w1b0, accd.at[pl.ds(rr, K)])
        plsc.subcore_barrier()

        ebase = wrk * EPW

        def idx_load(i, st):
            off = ebase + i * K
            pltpu.sync_copy(src_r.at[pl.ds(off, K)], st['sidx'])
            pltpu.sync_copy(dst_r.at[pl.ds(off, K)], st['didx'])

        def gather_descs(st):
            ds = [(tab_r.at[st['sidx']], st['rowb']),
                  (ss1_r.at[st['sidx']], st['w1b']),
                  (sd1_r.at[st['didx']], st['d1b'])]
            if two:
                ds += [(ss2_r.at[st['sidx']], st['w2b']),
                       (sd2_r.at[st['didx']], st['d2b'])]
            if has_mpnn:
                ds += [(xm_r.at[st['sidx']], st['mrow'])]
            return ds

        def scatter_descs(st):
            ds = [(st['rowb'], acc.at[st['didx']]),
                  (st['w1b'], accw1.at[st['didx']])]
            if two:
                ds += [(st['w2b'], accw2.at[st['didx']])]
            if has_mpnn:
                ds += [(st['mrow'], accm.at[st['didx']]),
                       (oneb, accd.at[st['didx']])]
            return ds

        def issue_gathers(st):
            for a, b in gather_descs(st):
                pltpu.async_copy(a, b, st['sem_g'])

        def drain_gathers(st):
            for a, b in gather_descs(st):
                pltpu.make_async_copy(a, b, st['sem_g']).wait()

        def issue_scatters(st):
            for a, b in scatter_descs(st):
                pltpu.async_copy(a, b, st['sem_sc'], add=True)

        def drain_scatters(st):
            for a, b in scatter_descs(st):
                pltpu.make_async_copy(a, b, st['sem_sc']).wait()

        def compute(st):
            rowb = st['rowb']; w1b = st['w1b']; d1b = st['d1b']
            if two:
                w2b = st['w2b']; d2b = st['d2b']

            def group(g, carry2):
                gb = g * 16
                z1 = w1b[pl.ds(gb, 16)] + d1b[pl.ds(gb, 16)]
                w1 = jnp.exp(jnp.where(z1 > 0, z1, z1 * 0.2))
                w1b[pl.ds(gb, 16)] = w1
                if two:
                    z2 = w2b[pl.ds(gb, 16)] + d2b[pl.ds(gb, 16)]
                    w2 = jnp.exp(jnp.where(z2 > 0, z2, z2 * 0.2))
                    w2b[pl.ds(gb, 16)] = w2
                for j in range(16):
                    e = gb + j
                    w1s = w1[j]
                    for b in range(F1 // 16):
                        rowb[e, pl.ds(b * 16, 16)] = rowb[e, pl.ds(b * 16, 16)] * w1s
                    if two:
                        w2s = w2[j]
                        for b in range(F2 // 16):
                            col = F1 + b * 16
                            rowb[e, pl.ds(col, 16)] = rowb[e, pl.ds(col, 16)] * w2s
                return carry2

            lax.fori_loop(0, K // 16, group, 0)

        idx_load(0, s0)
        issue_gathers(s0)

        def body2(j, carry):
            for b in range(nbuf):
                i = j * nbuf + b
                cur, nxt = sets[b], sets[b ^ 1]

                @pl.when(i >= 1)
                def _():
                    drain_scatters(nxt)

                @pl.when(i + 1 < NCHUNK)
                def _():
                    idx_load(i + 1, nxt)
                    issue_gathers(nxt)

                drain_gathers(cur)
                compute(cur)
                issue_scatters(cur)
            return carry

        lax.fori_loop(0, NCHUNK // nbuf, body2, 0)
        drain_scatters(sets[(NCHUNK - 1) % nbuf])
        plsc.subcore_barrier()

        mrow0 = s0['mrow'] if has_mpnn else None
        for i in range(nslab):
            rr = r0 + i * K
            pltpu.sync_copy(acc.at[pl.ds(rr, K)], rowb0)
            pltpu.sync_copy(rowb0, oacc_r.at[c, pl.ds(rr, K)])
            pltpu.sync_copy(accw1.at[pl.ds(rr, K)], w1b0)
            pltpu.sync_copy(w1b0, ow1_r.at[c, pl.ds(rr, K)])
            if two:
                pltpu.sync_copy(accw2.at[pl.ds(rr, K)], w1b0)
                pltpu.sync_copy(w1b0, ow2_r.at[c, pl.ds(rr, K)])
            if has_mpnn:
                pltpu.sync_copy(accd.at[pl.ds(rr, K)], w1b0)
                pltpu.sync_copy(w1b0, odeg_r.at[c, pl.ds(rr, K)])
                pltpu.sync_copy(accm.at[pl.ds(rr, K)], mrow0)
                pltpu.sync_copy(mrow0, om_r.at[c, pl.ds(rr, K)])

    return run


_make_edge_pass_cached = functools.cache(_make_edge_pass)


def _edge_pass1(*args):
    return _make_edge_pass_cached(F1=16, F2=16, has_mpnn=True)(*args)


def _edge_pass2(*args):
    return _make_edge_pass_cached(F1=64, F2=16, has_mpnn=False)(*args)


def _edge_pass3(*args):
    return _make_edge_pass_cached(F1=64, F2=0, has_mpnn=False)(*args)


def _leaky(z):
    return jnp.where(z > 0, z, z * 0.2)


def _tc_prep(x_r, w96_r, ag_r,
             tab_r, ssg_r, ssd_r, sdg_r, sdd_r, xm_r):
    h = jnp.dot(x_r[...], w96_r[...], preferred_element_type=_f32)
    hg = h[:, 0:16]
    hd = h[:, 16:32]
    ag = ag_r[...]
    tab_r[...] = h[:, 0:32]
    xm_r[...] = h[:, 32:96]
    ssg_r[...] = jnp.sum(hg * ag[0:1, :], axis=1)
    sdg_r[...] = jnp.sum(hg * ag[1:2, :], axis=1)
    ssd_r[...] = jnp.sum(hd * ag[2:3, :], axis=1)
    sdd_r[...] = jnp.sum(hd * ag[3:4, :], axis=1)


def _tc_mid_a(p1a_r, p1b_r, w1a_r, w1b_r, w2a_r, w2b_r, dega_r, degb_r,
              pma_r, pmb_r, tab1_r, ssg_r, sdg_r, ssd_r, sdd_r,
              bg1_r, bd1_r, wm2_r, bm_r,
              g_r, h0_r, mp_r):
    acc = p1a_r[...] + p1b_r[...]
    sw1 = w1a_r[...] + w1b_r[...]
    sw2 = w2a_r[...] + w2b_r[...]
    t1 = tab1_r[...]
    hg1 = t1[:, 0:16]
    hd1 = t1[:, 16:32]
    wg = jnp.exp(_leaky(ssg_r[...] + sdg_r[...]))
    wd = jnp.exp(_leaky(ssd_r[...] + sdd_r[...]))
    g1 = (acc[:, 0:16] + wg * hg1) / (sw1 + wg) + bg1_r[...]
    d1 = (acc[:, 16:32] + wd * hd1) / (sw2 + wd) + bd1_r[...]
    g_r[...] = jnp.maximum(g1, 0.0)
    h0_r[...] = d1
    deg = dega_r[...] + degb_r[...]

    accm = pma_r[...] + pmb_r[...]
    bm = bm_r[...]
    agg = accm + deg * bm[0:1, :]
    mp_r[...] = jnp.dot(agg, wm2_r[...], preferred_element_type=_f32) + bm[1:2, :]


def _tc_mid_b(g_r, h0f_r, h0_r, bn_r, wg2_r, a2_r, wd2_r,
              tab2_r, ssg2_r, ssd2_r, sdg2_r, sdd2_r):
    d1f = h0f_r[...]
    d1 = h0_r[...]
    rows = lax.broadcasted_iota(_i32, (NP, 1), 0)
    mask = (rows < N_NODES).astype(_f32)
    dm = d1f * mask
    inv_n = 1.0 / N_NODES
    mu = jnp.sum(dm, axis=0, keepdims=True) * inv_n
    ex2 = jnp.sum(dm * d1f, axis=0, keepdims=True) * inv_n
    var = ex2 - mu * mu
    bn = bn_r[...]
    dn = (d1 - mu) * jax.lax.rsqrt(var + 1e-5) * bn[0:1, :] + bn[1:2, :]
    dn = jnp.maximum(dn, 0.0)

    hg2 = jnp.dot(g_r[...], wg2_r[...], preferred_element_type=_f32)
    a2 = a2_r[...]
    hd2 = jnp.dot(dn, wd2_r[...], preferred_element_type=_f32)
    tab2_r[...] = jnp.concatenate([hg2, hd2], axis=1)
    ssg2_r[...] = jnp.sum(hg2 * a2[0:1, :], axis=1)
    sdg2_r[...] = jnp.sum(hg2 * a2[1:2, :], axis=1)
    ssd2_r[...] = jnp.sum(hd2 * a2[2:3, 0:16], axis=1)
    sdd2_r[...] = jnp.sum(hd2 * a2[3:4, 0:16], axis=1)


def _tc_mid2(p2a_r, p2b_r, w1a_r, w1b_r, w2a_r, w2b_r, tab2_r,
             ssg_r, sdg_r, ssd_r, sdd_r,
             bg2_r, h0_r, bd2_r, wd3_r, a3_r,
             g_r, tab3_r, ss3_r, sd3_r):
    acc = p2a_r[...] + p2b_r[...]
    sw1 = w1a_r[...] + w1b_r[...]
    sw2 = w2a_r[...] + w2b_r[...]
    t2 = tab2_r[...]
    hg2 = t2[:, 0:64]
    hd2 = t2[:, 64:80]
    wg = jnp.exp(_leaky(ssg_r[...] + sdg_r[...]))
    wd = jnp.exp(_leaky(ssd_r[...] + sdd_r[...]))
    g_r[...] = (acc[:, 0:64] + wg * hg2) / (sw1 + wg) + bg2_r[...]
    d2 = (acc[:, 64:80] + wd * hd2) / (sw2 + wd) + bd2_r[...] + h0_r[...]

    hd3 = jnp.dot(d2, wd3_r[...], preferred_element_type=_f32)
    a3 = a3_r[...]
    tab3_r[...] = hd3
    ss3_r[...] = jnp.sum(hd3 * a3[0:1, :], axis=1)
    sd3_r[...] = jnp.sum(hd3 * a3[1:2, :], axis=1)


def _tc_fin(p3a_r, p3b_r, w1a_r, w1b_r, tab3_r, ss3_r, sd3_r, bd3_r, d_r):
    acc = p3a_r[...] + p3b_r[...]
    sw = w1a_r[...] + w1b_r[...]
    h3 = tab3_r[...]
    w = jnp.exp(_leaky(ss3_r[...] + sd3_r[...]))
    d_r[...] = (acc + w * h3) / (sw + w) + bd3_r[...]


_BLK = 2048
_NB = NP // _BLK


def _tc_pool(g_r, d_r, mp_r, batch_r, out_r):
    i = pl.program_id(0)
    gids = lax.broadcasted_iota(_i32, (N_GRAPHS, _BLK), 0)
    P = (gids == batch_r[...]).astype(_f32)
    part = jnp.concatenate([
        jnp.dot(P, g_r[...], preferred_element_type=_f32),
        jnp.dot(P, d_r[...], preferred_element_type=_f32),
        jnp.dot(P, mp_r[...], preferred_element_type=_f32),
        jnp.sum(P, axis=1, keepdims=True),
        jnp.zeros((N_GRAPHS, 63), _f32)], axis=1)

    @pl.when(i == 0)
    def _():
        out_r[...] = jnp.zeros((N_GRAPHS, 256), _f32)

    out_r[...] += part


def _tc_head(pool_r, cls_r, mean_r, wc1_r, wc2_r, wm1_r, wm2_r, wf1_r,
             wf2_r, wa1_r, wa2_r, bvec_r, out_r):
    pool = pool_r[...]
    cnt = jnp.maximum(pool[:, 192:193], 1.0)
    gp = pool[:, 0:64] / cnt
    dp = pool[:, 64:128] / cnt
    mpp = pool[:, 128:192] / cnt

    bv = bvec_r[...]
    c = jnp.maximum(jnp.dot(cls_r[...], wc1_r[...], preferred_element_type=_f32)
                    + bv[0:1, 0:128], 0.0)
    c = jnp.dot(c, wc2_r[...], preferred_element_type=_f32) + bv[1:2, 0:64]
    me = jnp.maximum(jnp.dot(mean_r[...], wm1_r[...], preferred_element_type=_f32)
                     + bv[2:3, 0:128], 0.0)
    me = jnp.dot(me, wm2_r[...], preferred_element_type=_f32) + bv[3:4, 0:64]
    emb = jnp.maximum(jnp.dot(jnp.concatenate([c, me], axis=1), wf1_r[...],
                              preferred_element_type=_f32) + bv[4:5, 0:32], 0.0)
    emb = jnp.dot(emb, wf2_r[...], preferred_element_type=_f32) + bv[5:6, 0:64]

    feat = jnp.concatenate([gp, dp, mpp, emb], axis=1)
    o = jnp.maximum(jnp.dot(feat, wa1_r[...], preferred_element_type=_f32)
                    + bv[6:7, 0:64], 0.0)
    out_r[...] = jnp.dot(o, wa2_r[...], preferred_element_type=_f32) + bv[7:8, 0:1]


_PALLAS_CALL = pl.pallas_call


def _spec(kind, shape):
    nd = len(shape)
    if kind == 'R':
        return pl.BlockSpec((_BLK,) + tuple(shape[1:]),
                            lambda i: (i,) + (0,) * (nd - 1))
    if kind == 'r':
        return pl.BlockSpec((_BLK,), lambda i: (i,))
    if kind == 'b':
        return pl.BlockSpec((1, _BLK), lambda i: (0, i))
    return pl.BlockSpec(tuple(shape), lambda i: (0,) * nd)


def _tc_call(body, outs, args, grid=None):
    if grid is None:
        return _PALLAS_CALL(
            body,
            out_shape=tuple(jax.ShapeDtypeStruct(sh, _f32) for sh, _ in outs),
        )(*[v for v, _ in args])
    return _PALLAS_CALL(
        body,
        grid=(grid,),
        in_specs=[_spec(k, v.shape) for v, k in args],
        out_specs=tuple(_spec(k, sh) for sh, k in outs),
        out_shape=tuple(jax.ShapeDtypeStruct(sh, _f32) for sh, _ in outs),
    )(*[v for v, _ in args])


def kernel(x, edge_index, batch, cls_embed, mean_embed, params):
    p = params
    xp = jnp.pad(x.astype(_f32), ((0, NP - N_NODES), (0, 0)))
    pad_e = EP - edge_index.shape[1]
    src = jnp.concatenate([edge_index[0].astype(_i32),
                           jnp.full((pad_e,), N_NODES, _i32)])
    dst = jnp.concatenate([edge_index[1].astype(_i32),
                           jnp.full((pad_e,), N_NODES, _i32)])
    batch_p = jnp.pad(batch.astype(_i32), (0, NP - N_NODES),
                      constant_values=N_GRAPHS).reshape(1, NP)

    w96 = jnp.concatenate([p['W_g1'], p['W_d1'], p['W_m1']], axis=1)
    ag = jnp.stack([p['as_g1'], p['ad_g1'], p['as_d1'], p['ad_d1']])
    pad16 = jnp.zeros((48,), _f32)
    a2 = jnp.stack([p['as_g2'], p['ad_g2'],
                    jnp.concatenate([p['as_d2'], pad16]),
                    jnp.concatenate([p['ad_d2'], pad16])])
    a3 = jnp.stack([p['as_d3'], p['ad_d3']])
    bm = jnp.stack([p['b_m1'], p['b_m2']])
    bn = jnp.stack([p['bn_g'], p['bn_b']])

    def row128(v):
        return jnp.pad(v, (0, 128 - v.shape[0]))
    bvec = jnp.stack([row128(p['bc1']), row128(p['bc2']), row128(p['bm1']),
                      row128(p['bm2']), row128(p['bf1']), row128(p['bf2']),
                      row128(p['ba1']), row128(p['ba2'])])

    def col(v):
        return v.reshape(NP, 1)

    tab1, ssg1, ssd1, sdg1, sdd1, xm = _tc_call(
        _tc_prep,
        [((NP, 32), 'R'), ((NP,), 'r'), ((NP,), 'r'), ((NP,), 'r'),
         ((NP,), 'r'), ((NP, 64), 'R')],
        [(xp, 'R'), (w96, 'F'), (ag, 'F')], grid=_NB)

    acc1, ww1, ww2, wdeg, accm = _edge_pass1(
        src, dst, tab1, ssg1, sdg1, ssd1, sdd1, xm)

    g, h0, mp = _tc_call(
        _tc_mid_a, [((NP, 16), 'R'), ((NP, 16), 'R'), ((NP, 64), 'R')],
        [(acc1[0], 'R'), (acc1[1], 'R'),
         (col(ww1[0]), 'R'), (col(ww1[1]), 'R'),
         (col(ww2[0]), 'R'), (col(ww2[1]), 'R'),
         (col(wdeg[0]), 'R'), (col(wdeg[1]), 'R'),
         (accm[0], 'R'), (accm[1], 'R'), (tab1, 'R'),
         (col(ssg1), 'R'), (col(sdg1), 'R'), (col(ssd1), 'R'),
         (col(sdd1), 'R'),
         (p['b_g1'].reshape(1, 16), 'F'), (p['b_d1'].reshape(1, 16), 'F'),
         (p['W_m2'], 'F'), (bm, 'F')], grid=_NB)
    tab2, ssg2, ssd2, sdg2, sdd2 = _tc_call(
        _tc_mid_b,
        [((NP, 80), 'R'), ((NP,), 'r'), ((NP,), 'r'), ((NP,), 'r'),
         ((NP,), 'r')],
        [(g, 'R'), (h0, 'F'), (h0, 'R'), (bn, 'F'), (p['W_g2'], 'F'),
         (a2, 'F'), (p['W_d2'], 'F')], grid=_NB)

    acc2, w21, w22 = _edge_pass2(src, dst, tab2, ssg2, sdg2, ssd2, sdd2)

    gfin, tab3, ss3, sd3 = _tc_call(
        _tc_mid2,
        [((NP, 64), 'R'), ((NP, 64), 'R'), ((NP,), 'r'), ((NP,), 'r')],
        [(acc2[0], 'R'), (acc2[1], 'R'),
         (col(w21[0]), 'R'), (col(w21[1]), 'R'),
         (col(w22[0]), 'R'), (col(w22[1]), 'R'), (tab2, 'R'),
         (col(ssg2), 'R'), (col(sdg2), 'R'), (col(ssd2), 'R'),
         (col(sdd2), 'R'),
         (p['b_g2'].reshape(1, 64), 'F'), (h0, 'R'),
         (p['b_d2'].reshape(1, 16), 'F'), (p['W_d3'], 'F'),
         (a3, 'F')], grid=_NB)

    acc3, w31 = _edge_pass3(src, dst, tab3, ss3, sd3)

    dfin, = _tc_call(
        _tc_fin, [((NP, 64), 'R')],
        [(acc3[0], 'R'), (acc3[1], 'R'),
         (col(w31[0]), 'R'), (col(w31[1]), 'R'), (tab3, 'R'),
         (col(ss3), 'R'), (col(sd3), 'R'),
         (p['b_d3'].reshape(1, 64), 'F')], grid=_NB)
    pool, = _tc_call(
        _tc_pool, [((N_GRAPHS, 256), 'F')],
        [(gfin, 'R'), (dfin, 'R'), (mp, 'R'), (batch_p, 'b')], grid=_NB)
    out, = _tc_call(
        _tc_head, [((N_GRAPHS, 1), 'F')],
        [(pool, 'F'), (cls_embed, 'F'), (mean_embed, 'F'),
         (p['Wc1'], 'F'), (p['Wc2'], 'F'), (p['Wm1'], 'F'), (p['Wm2'], 'F'),
         (p['Wf1'], 'F'), (p['Wf2'], 'F'), (p['Wa1'], 'F'), (p['Wa2'], 'F'),
         (bvec, 'F')])
    return out
